# Initial kernel scaffold; baseline (speedup 1.0000x reference)
import functools

import jax
import jax.numpy as jnp
from jax import lax
from jax.experimental import pallas as pl
from jax.experimental.pallas import tpu as pltpu
from jax.experimental.pallas import tpu_sc as plsc

N = 10000
E = 320000
G = 64
D = 128
DO = 64

NC = 2
NS = 16
EC = E // NC
ET = EC // NS
K = 80
NCHUNK = ET // K
RPT = N // NS
ZR = 125

_mesh = plsc.VectorSubcoreMesh(core_axis_name="c", subcore_axis_name="s")


@functools.partial(
    pl.kernel,
    mesh=_mesh,
    out_type=jax.ShapeDtypeStruct((NC, N, 16), jnp.float32),
    scratch_types=[
        pltpu.VMEM_SHARED((N, 16), jnp.float32),
        pltpu.VMEM((ZR, 16), jnp.float32),
        pltpu.VMEM((K, 16), jnp.float32),
        pltpu.VMEM((K,), jnp.int32),
    ],
)
def _sc_degree(dst_hbm, out_hbm, acc_sh, zbuf, ones_v, idx_v):
    c = lax.axis_index("c")
    s = lax.axis_index("s")

    @pl.loop(0, ZR)
    def _(r):
        zbuf[r, :] = jnp.zeros((16,), jnp.float32)

    @pl.loop(0, K)
    def _(r):
        ones_v[r, :] = jnp.ones((16,), jnp.float32)

    @pl.loop(0, RPT // ZR)
    def _(j):
        pltpu.sync_copy(zbuf, acc_sh.at[pl.ds(s * RPT + j * ZR, ZR)])

    plsc.subcore_barrier()

    base0 = c * EC + s * ET

    @pl.loop(0, NCHUNK)
    def _(i):
        pltpu.sync_copy(dst_hbm.at[pl.ds(base0 + i * K, K)], idx_v)
        pltpu.sync_copy(ones_v, acc_sh.at[idx_v], add=True)

    plsc.subcore_barrier()
    pltpu.sync_copy(acc_sh.at[pl.ds(s * RPT, RPT)],
                    out_hbm.at[c].at[pl.ds(s * RPT, RPT)])


@functools.partial(
    pl.kernel,
    mesh=_mesh,
    out_type=jax.ShapeDtypeStruct((NC, N, D), jnp.float32),
    scratch_types=[
        pltpu.VMEM_SHARED((N, D), jnp.float32),
        pltpu.VMEM((ZR, D), jnp.float32),
        pltpu.VMEM((K, D), jnp.float32),
        pltpu.VMEM((K,), jnp.int32),
        pltpu.VMEM((K,), jnp.int32),
    ],
)
def _sc_aggregate(hs_hbm, src_hbm, dst_hbm, out_hbm,
                  acc_sh, zbuf, rows_v, sidx_v, didx_v):
    c = lax.axis_index("c")
    s = lax.axis_index("s")

    @pl.loop(0, ZR)
    def _(r):
        @pl.loop(0, D // 16)
        def _(j):
            zbuf[r, pl.ds(j * 16, 16)] = jnp.zeros((16,), jnp.float32)

    @pl.loop(0, RPT // ZR)
    def _(j):
        pltpu.sync_copy(zbuf, acc_sh.at[pl.ds(s * RPT + j * ZR, ZR)])

    plsc.subcore_barrier()

    base0 = c * EC + s * ET

    @pl.loop(0, NCHUNK)
    def _(i):
        pltpu.sync_copy(src_hbm.at[pl.ds(base0 + i * K, K)], sidx_v)
        pltpu.sync_copy(dst_hbm.at[pl.ds(base0 + i * K, K)], didx_v)
        pltpu.sync_copy(hs_hbm.at[sidx_v], rows_v)
        pltpu.sync_copy(rows_v, acc_sh.at[didx_v], add=True)

    plsc.subcore_barrier()
    pltpu.sync_copy(acc_sh.at[pl.ds(s * RPT, RPT)],
                    out_hbm.at[c].at[pl.ds(s * RPT, RPT)])


BLK = 1000
GRID = N // BLK


def _dinv_of(da_ref, db_ref):
    deg = da_ref[:, :1] + db_ref[:, :1] + 1.0
    return lax.rsqrt(deg)


def _tc1_body(x_ref, w1_ref, da_ref, db_ref, hs_ref):
    dinv = _dinv_of(da_ref, db_ref)
    h = jnp.dot(x_ref[...], w1_ref[...], preferred_element_type=jnp.float32)
    hs_ref[...] = h * dinv


def _tc2_body(aa_ref, ab_ref, hs1_ref, da_ref, db_ref, b1_ref, w2_ref,
              hs2_ref):
    dinv = _dinv_of(da_ref, db_ref)
    conv1 = dinv * (aa_ref[...] + ab_ref[...] + hs1_ref[...]) + b1_ref[...]
    t = jnp.maximum(conv1, 0.0)
    h2 = jnp.dot(t, w2_ref[...], preferred_element_type=jnp.float32)
    hs2_ref[...] = h2 * dinv


def _tc3_body(aa_ref, ab_ref, hs2_ref, da_ref, db_ref, b2_ref, w3_ref,
              b3_ref, batch_ref, out_ref, cnt_ref):
    i = pl.program_id(0)
    dinv = _dinv_of(da_ref, db_ref)
    conv2 = dinv * (aa_ref[...] + ab_ref[...] + hs2_ref[...]) + b2_ref[...]
    t = jnp.maximum(conv2, 0.0)
    h3 = jnp.dot(t, w3_ref[...], preferred_element_type=jnp.float32)
    h3 = h3 + b3_ref[...]

    bb = batch_ref[0, 0, :]
    gids = lax.broadcasted_iota(jnp.int32, (BLK, G), 1)
    p = (bb[:, None] == gids).astype(jnp.float32)
    dn = (((0,), (0,)), ((), ()))
    partial = lax.dot_general(p, h3, dn, preferred_element_type=jnp.float32)
    ones_col = jnp.ones((BLK, 1), jnp.float32)
    cnt = lax.dot_general(p, ones_col, dn, preferred_element_type=jnp.float32)

    @pl.when(i == 0)
    def _():
        out_ref[...] = partial
        cnt_ref[...] = cnt

    @pl.when(i > 0)
    def _():
        out_ref[...] += partial
        cnt_ref[...] += cnt

    @pl.when(i == GRID - 1)
    def _():
        out_ref[...] = out_ref[...] / jnp.maximum(cnt_ref[...], 1.0)


def _row_spec(width):
    return pl.BlockSpec((BLK, width), lambda i: (i, 0))


def _full_spec(shape):
    nd = len(shape)
    return pl.BlockSpec(shape, lambda i: (0,) * nd)


_tc1 = pl.pallas_call(
    _tc1_body,
    grid=(GRID,),
    in_specs=[_row_spec(D), _full_spec((D, D)), _row_spec(16), _row_spec(16)],
    out_specs=_row_spec(D),
    out_shape=jax.ShapeDtypeStruct((N, D), jnp.float32),
)

_tc2 = pl.pallas_call(
    _tc2_body,
    grid=(GRID,),
    in_specs=[_row_spec(D), _row_spec(D), _row_spec(D), _row_spec(16),
              _row_spec(16), _full_spec((1, D)), _full_spec((D, D))],
    out_specs=_row_spec(D),
    out_shape=jax.ShapeDtypeStruct((N, D), jnp.float32),
)

_tc3 = pl.pallas_call(
    _tc3_body,
    grid=(GRID,),
    in_specs=[_row_spec(D), _row_spec(D), _row_spec(D), _row_spec(16),
              _row_spec(16), _full_spec((1, D)), _full_spec((D, DO)),
              _full_spec((1, DO)),
              pl.BlockSpec((1, 1, BLK), lambda i: (i, 0, 0))],
    out_specs=_full_spec((G, DO)),
    out_shape=jax.ShapeDtypeStruct((G, DO), jnp.float32),
    scratch_shapes=[pltpu.VMEM((G, 1), jnp.float32)],
)


def kernel(x, edge_index, batch, W1, b1, W2, b2, W3, b3):
    src = edge_index[0]
    dst = edge_index[1]
    b1r = b1.reshape(1, D)
    b2r = b2.reshape(1, D)
    b3r = b3.reshape(1, DO)
    batch3 = batch.reshape(GRID, 1, BLK)

    degp = _sc_degree(dst)
    da = degp[0]
    db = degp[1]

    hs1 = _tc1(x, W1, da, db)
    agg1 = _sc_aggregate(hs1, src, dst)
    hs2 = _tc2(agg1[0], agg1[1], hs1, da, db, b1r, W2)
    agg2 = _sc_aggregate(hs2, src, dst)
    out = _tc3(agg2[0], agg2[1], hs2, da, db, b2r, W3, b3r, batch3)
    return out


# speedup vs baseline: 12.8538x; 12.8538x over previous
import functools

import jax
import jax.numpy as jnp
from jax import lax
from jax.experimental import pallas as pl
from jax.experimental.pallas import tpu as pltpu
from jax.experimental.pallas import tpu_sc as plsc

N = 10000
E = 320000
G = 64
D = 128
DO = 64

NC = 2
NS = 16
EC = E // NC
ET = EC // NS
K = 80
NCHUNK = ET // K
NP = 10240
RPT = NP // NS
ZR = 128

_mesh = plsc.VectorSubcoreMesh(core_axis_name="c", subcore_axis_name="s")


@functools.partial(
    pl.kernel,
    mesh=_mesh,
    out_type=jax.ShapeDtypeStruct((NC, NP, 16), jnp.float32),
    scratch_types=[
        pltpu.VMEM_SHARED((NP, 16), jnp.float32),
        pltpu.VMEM((ZR, 16), jnp.float32),
        pltpu.VMEM((K, 16), jnp.float32),
        pltpu.VMEM((K,), jnp.int32),
    ],
)
def _sc_degree(dst_hbm, out_hbm, acc_sh, zbuf, ones_v, idx_v):
    c = lax.axis_index("c")
    s = lax.axis_index("s")

    @pl.loop(0, ZR)
    def _(r):
        zbuf[r, :] = jnp.zeros((16,), jnp.float32)

    @pl.loop(0, K)
    def _(r):
        ones_v[r, :] = jnp.ones((16,), jnp.float32)

    @pl.loop(0, RPT // ZR)
    def _(j):
        pltpu.sync_copy(zbuf, acc_sh.at[pl.ds(s * RPT + j * ZR, ZR)])

    plsc.subcore_barrier()

    base0 = c * EC + s * ET

    @pl.loop(0, NCHUNK)
    def _(i):
        pltpu.sync_copy(dst_hbm.at[pl.ds(base0 + i * K, K)], idx_v)
        pltpu.sync_copy(ones_v, acc_sh.at[idx_v], add=True)

    plsc.subcore_barrier()
    pltpu.sync_copy(acc_sh.at[pl.ds(s * RPT, RPT)],
                    out_hbm.at[c].at[pl.ds(s * RPT, RPT)])


@functools.partial(
    pl.kernel,
    mesh=_mesh,
    out_type=jax.ShapeDtypeStruct((NC, NP, D), jnp.float32),
    scratch_types=[
        pltpu.VMEM_SHARED((NP, D), jnp.float32),
        pltpu.VMEM((ZR, D), jnp.float32),
        pltpu.VMEM((K, D), jnp.float32),
        pltpu.VMEM((K,), jnp.int32),
        pltpu.VMEM((K,), jnp.int32),
    ],
)
def _sc_aggregate(hs_hbm, src_hbm, dst_hbm, out_hbm,
                  acc_sh, zbuf, rows_v, sidx_v, didx_v):
    c = lax.axis_index("c")
    s = lax.axis_index("s")

    @pl.loop(0, ZR)
    def _(r):
        @pl.loop(0, D // 16)
        def _(j):
            zbuf[r, pl.ds(j * 16, 16)] = jnp.zeros((16,), jnp.float32)

    @pl.loop(0, RPT // ZR)
    def _(j):
        pltpu.sync_copy(zbuf, acc_sh.at[pl.ds(s * RPT + j * ZR, ZR)])

    plsc.subcore_barrier()

    base0 = c * EC + s * ET

    @pl.loop(0, NCHUNK)
    def _(i):
        pltpu.sync_copy(src_hbm.at[pl.ds(base0 + i * K, K)], sidx_v)
        pltpu.sync_copy(dst_hbm.at[pl.ds(base0 + i * K, K)], didx_v)
        pltpu.sync_copy(hs_hbm.at[sidx_v], rows_v)
        pltpu.sync_copy(rows_v, acc_sh.at[didx_v], add=True)

    plsc.subcore_barrier()
    pltpu.sync_copy(acc_sh.at[pl.ds(s * RPT, RPT)],
                    out_hbm.at[c].at[pl.ds(s * RPT, RPT)])


BLK = 1000
GRID = N // BLK


def _dinv_of(da_ref, db_ref):
    deg = da_ref[:, :1] + db_ref[:, :1] + 1.0
    return lax.rsqrt(deg)


def _tc1_body(x_ref, w1_ref, da_ref, db_ref, hs_ref):
    dinv = _dinv_of(da_ref, db_ref)
    h = jnp.dot(x_ref[...], w1_ref[...], preferred_element_type=jnp.float32)
    hs_ref[...] = h * dinv


def _tc2_body(aa_ref, ab_ref, hs1_ref, da_ref, db_ref, b1_ref, w2_ref,
              hs2_ref):
    dinv = _dinv_of(da_ref, db_ref)
    conv1 = dinv * (aa_ref[...] + ab_ref[...] + hs1_ref[...]) + b1_ref[...]
    t = jnp.maximum(conv1, 0.0)
    h2 = jnp.dot(t, w2_ref[...], preferred_element_type=jnp.float32)
    hs2_ref[...] = h2 * dinv


def _tc3_body(aa_ref, ab_ref, hs2_ref, da_ref, db_ref, b2_ref, w3_ref,
              b3_ref, batch_ref, out_ref, cnt_ref):
    i = pl.program_id(0)
    dinv = _dinv_of(da_ref, db_ref)
    conv2 = dinv * (aa_ref[...] + ab_ref[...] + hs2_ref[...]) + b2_ref[...]
    t = jnp.maximum(conv2, 0.0)
    h3 = jnp.dot(t, w3_ref[...], preferred_element_type=jnp.float32)
    h3 = h3 + b3_ref[...]

    bb = batch_ref[0, 0, :]
    gids = lax.broadcasted_iota(jnp.int32, (BLK, G), 1)
    p = (bb[:, None] == gids).astype(jnp.float32)
    dn = (((0,), (0,)), ((), ()))
    partial = lax.dot_general(p, h3, dn, preferred_element_type=jnp.float32)
    ones_col = jnp.ones((BLK, 1), jnp.float32)
    cnt = lax.dot_general(p, ones_col, dn, preferred_element_type=jnp.float32)

    @pl.when(i == 0)
    def _():
        out_ref[...] = partial
        cnt_ref[...] = cnt

    @pl.when(i > 0)
    def _():
        out_ref[...] += partial
        cnt_ref[...] += cnt

    @pl.when(i == GRID - 1)
    def _():
        out_ref[...] = out_ref[...] / jnp.maximum(cnt_ref[...], 1.0)


def _row_spec(width):
    return pl.BlockSpec((BLK, width), lambda i: (i, 0))


def _full_spec(shape):
    nd = len(shape)
    return pl.BlockSpec(shape, lambda i: (0,) * nd)


_tc1 = pl.pallas_call(
    _tc1_body,
    grid=(GRID,),
    in_specs=[_row_spec(D), _full_spec((D, D)), _row_spec(16), _row_spec(16)],
    out_specs=_row_spec(D),
    out_shape=jax.ShapeDtypeStruct((N, D), jnp.float32),
)

_tc2 = pl.pallas_call(
    _tc2_body,
    grid=(GRID,),
    in_specs=[_row_spec(D), _row_spec(D), _row_spec(D), _row_spec(16),
              _row_spec(16), _full_spec((1, D)), _full_spec((D, D))],
    out_specs=_row_spec(D),
    out_shape=jax.ShapeDtypeStruct((N, D), jnp.float32),
)

_tc3 = pl.pallas_call(
    _tc3_body,
    grid=(GRID,),
    in_specs=[_row_spec(D), _row_spec(D), _row_spec(D), _row_spec(16),
              _row_spec(16), _full_spec((1, D)), _full_spec((D, DO)),
              _full_spec((1, DO)),
              pl.BlockSpec((1, 1, BLK), lambda i: (i, 0, 0))],
    out_specs=_full_spec((G, DO)),
    out_shape=jax.ShapeDtypeStruct((G, DO), jnp.float32),
    scratch_shapes=[pltpu.VMEM((G, 1), jnp.float32)],
)


def kernel(x, edge_index, batch, W1, b1, W2, b2, W3, b3):
    src = edge_index[0]
    dst = edge_index[1]
    b1r = b1.reshape(1, D)
    b2r = b2.reshape(1, D)
    b3r = b3.reshape(1, DO)
    batch3 = batch.reshape(GRID, 1, BLK)

    degp = _sc_degree(dst)
    da = degp[0, :N]
    db = degp[1, :N]

    hs1 = _tc1(x, W1, da, db)
    agg1 = _sc_aggregate(hs1, src, dst)
    hs2 = _tc2(agg1[0, :N], agg1[1, :N], hs1, da, db, b1r, W2)
    agg2 = _sc_aggregate(hs2, src, dst)
    out = _tc3(agg2[0, :N], agg2[1, :N], hs2, da, db, b2r, W3, b3r, batch3)
    return out
